# Initial kernel scaffold; baseline (speedup 1.0000x reference)
import jax
import jax.numpy as jnp
from jax.experimental import pallas as pl


def kernel(edge_index, edge_attr, N):
    raise NotImplementedError("write your pallas kernel here")


# speedup vs baseline: 101.5626x; 101.5626x over previous
import functools

import jax
import jax.numpy as jnp
from jax import lax
from jax.experimental import pallas as pl
from jax.experimental.pallas import tpu as pltpu
from jax.experimental.pallas import tpu_sc as plsc

N_STATIC = 50000
E_STATIC = 1600000
LANES = 128
ROWS = E_STATIC // LANES
NW = 32
BR = 16
WROWS = 400
NB = WROWS // BR
NB_LAST = 6
TAIL_BASE = 31 * WROWS + NB_LAST * BR
TAIL_ROWS = ROWS - TAIL_BASE
NSLICE = 16
SLICE = 3136
NPAD = NSLICE * SLICE


def _rsqrt_nr(d):
    i = lax.bitcast_convert_type(d, jnp.int32)
    y = lax.bitcast_convert_type(jnp.int32(0x5F3759DF) - (i >> 1), jnp.float32)
    half = d * jnp.float32(0.5)
    for _ in range(4):
        y = y * (jnp.float32(1.5) - half * y * y)
    return jnp.where(d > jnp.float32(0.0), y, jnp.float32(jnp.inf))


def _mesh():
    return plsc.VectorSubcoreMesh(core_axis_name="c", subcore_axis_name="s")


@functools.partial(
    pl.kernel,
    mesh=_mesh(),
    compiler_params=pltpu.CompilerParams(needs_layout_passes=False),
    out_type=(jax.ShapeDtypeStruct((NPAD,), jnp.float32),
              jax.ShapeDtypeStruct((NPAD,), jnp.float32)),
    scratch_types=[
        pltpu.VMEM((BR, LANES), jnp.int32),
        pltpu.VMEM((BR, LANES), jnp.float32),
        pltpu.VMEM((SLICE,), jnp.float32),
        pltpu.VMEM_SHARED((NPAD,), jnp.float32),
    ],
)
def _deg_kernel(src2, attr2, dp0, dp1, idx_v, val_v, slice_v, deg_sh):
    c = lax.axis_index("c")
    s = lax.axis_index("s")
    w = s * 2 + c
    sl = pl.ds(s * SLICE, SLICE)

    def zero_vec(i, carry):
        slice_v[pl.ds(i * 16, 16)] = jnp.zeros((16,), jnp.float32)
        return carry

    lax.fori_loop(0, SLICE // 16, zero_vec, 0)
    pltpu.sync_copy(slice_v, deg_sh.at[sl])
    plsc.subcore_barrier()

    def blk(b, carry):
        rb = w * WROWS + b * BR
        pltpu.sync_copy(src2.at[pl.ds(rb, BR)], idx_v)
        pltpu.sync_copy(attr2.at[pl.ds(rb, BR)], val_v)
        for j in range(BR):
            pltpu.sync_copy(val_v.at[j], deg_sh.at[idx_v.at[j]], add=True)
        return carry

    nb = jnp.where(w == NW - 1, NB_LAST, NB)
    lax.fori_loop(0, nb, blk, 0)

    @pl.when(w == NW - 1)
    def _tail():
        pltpu.sync_copy(src2.at[pl.ds(TAIL_BASE, TAIL_ROWS)],
                        idx_v.at[pl.ds(0, TAIL_ROWS)])
        pltpu.sync_copy(attr2.at[pl.ds(TAIL_BASE, TAIL_ROWS)],
                        val_v.at[pl.ds(0, TAIL_ROWS)])
        for j in range(TAIL_ROWS):
            pltpu.sync_copy(val_v.at[j], deg_sh.at[idx_v.at[j]], add=True)

    plsc.subcore_barrier()
    pltpu.sync_copy(deg_sh.at[sl], slice_v)

    @pl.when(c == 0)
    def _w0():
        pltpu.sync_copy(slice_v, dp0.at[sl])

    @pl.when(c == 1)
    def _w1():
        pltpu.sync_copy(slice_v, dp1.at[sl])


@functools.partial(
    pl.kernel,
    mesh=_mesh(),
    compiler_params=pltpu.CompilerParams(needs_layout_passes=False),
    out_type=jax.ShapeDtypeStruct((ROWS, LANES), jnp.float32),
    scratch_types=[
        pltpu.VMEM((BR, LANES), jnp.int32),
        pltpu.VMEM((BR, LANES), jnp.int32),
        pltpu.VMEM((BR, LANES), jnp.float32),
        pltpu.VMEM((BR, LANES), jnp.float32),
        pltpu.VMEM((SLICE,), jnp.float32),
        pltpu.VMEM((SLICE,), jnp.float32),
        pltpu.VMEM((NPAD,), jnp.float32),
        pltpu.VMEM_SHARED((NPAD,), jnp.float32),
    ],
)
def _norm_kernel(src2, dst2, attr2, dp0, dp1, out2,
                 si_v, di_v, at_v, out_v, a_v, b_v, rn_full, rn_sh):
    c = lax.axis_index("c")
    s = lax.axis_index("s")
    w = s * 2 + c
    sl = pl.ds(s * SLICE, SLICE)

    pltpu.sync_copy(dp0.at[sl], a_v)
    pltpu.sync_copy(dp1.at[sl], b_v)

    def rn_vec(i, carry):
        v = pl.ds(i * 16, 16)
        a_v[v] = _rsqrt_nr(a_v[v] + b_v[v])
        return carry

    lax.fori_loop(0, SLICE // 16, rn_vec, 0)
    pltpu.sync_copy(a_v, rn_sh.at[sl])
    plsc.subcore_barrier()
    pltpu.sync_copy(rn_sh, rn_full)

    def compute_rows(nrows):
        for j in range(nrows):
            for k in range(LANES // 16):
                v = pl.ds(k * 16, 16)
                r = plsc.load_gather(rn_full, [si_v[j, v]])
                col = plsc.load_gather(rn_full, [di_v[j, v]])
                out_v[j, v] = r * at_v[j, v] * col

    def blk(b, carry):
        rb = w * WROWS + b * BR
        pltpu.sync_copy(src2.at[pl.ds(rb, BR)], si_v)
        pltpu.sync_copy(dst2.at[pl.ds(rb, BR)], di_v)
        pltpu.sync_copy(attr2.at[pl.ds(rb, BR)], at_v)
        compute_rows(BR)
        pltpu.sync_copy(out_v, out2.at[pl.ds(rb, BR)])
        return carry

    nb = jnp.where(w == NW - 1, NB_LAST, NB)
    lax.fori_loop(0, nb, blk, 0)

    @pl.when(w == NW - 1)
    def _tail():
        pltpu.sync_copy(src2.at[pl.ds(TAIL_BASE, TAIL_ROWS)],
                        si_v.at[pl.ds(0, TAIL_ROWS)])
        pltpu.sync_copy(dst2.at[pl.ds(TAIL_BASE, TAIL_ROWS)],
                        di_v.at[pl.ds(0, TAIL_ROWS)])
        pltpu.sync_copy(attr2.at[pl.ds(TAIL_BASE, TAIL_ROWS)],
                        at_v.at[pl.ds(0, TAIL_ROWS)])
        compute_rows(TAIL_ROWS)
        pltpu.sync_copy(out_v.at[pl.ds(0, TAIL_ROWS)],
                        out2.at[pl.ds(TAIL_BASE, TAIL_ROWS)])


def kernel(edge_index, edge_attr, N):
    src2 = edge_index[0].astype(jnp.int32).reshape(ROWS, LANES)
    dst2 = edge_index[1].astype(jnp.int32).reshape(ROWS, LANES)
    attr2 = edge_attr.reshape(ROWS, LANES)
    dp0, dp1 = _deg_kernel(src2, attr2)
    out2 = _norm_kernel(src2, dst2, attr2, dp0, dp1)
    return out2.reshape(E_STATIC)


# speedup vs baseline: 113.3915x; 1.1165x over previous
import functools

import jax
import jax.numpy as jnp
from jax import lax
from jax.experimental import pallas as pl
from jax.experimental.pallas import tpu as pltpu
from jax.experimental.pallas import tpu_sc as plsc

N_STATIC = 50000
E_STATIC = 1600000
LANES = 128
ROWS = E_STATIC // LANES
NW = 32
BR = 16
WROWS = 400
NB = WROWS // BR
NB_LAST = 6
TAIL_BASE = 31 * WROWS + NB_LAST * BR
TAIL_ROWS = ROWS - TAIL_BASE
NSLICE = 16
SLICE = 3200
NPAD = NSLICE * SLICE


def _rsqrt_nr(d):
    i = lax.bitcast_convert_type(d, jnp.int32)
    y = lax.bitcast_convert_type(jnp.int32(0x5F3759DF) - (i >> 1), jnp.float32)
    half = d * jnp.float32(0.5)
    for _ in range(4):
        y = y * (jnp.float32(1.5) - half * y * y)
    return jnp.where(d > jnp.float32(0.0), y, jnp.float32(jnp.inf))


def _mesh():
    return plsc.VectorSubcoreMesh(core_axis_name="c", subcore_axis_name="s")


_PARAMS = pltpu.CompilerParams(needs_layout_passes=False)


@functools.partial(
    pl.kernel,
    mesh=_mesh(),
    compiler_params=_PARAMS,
    out_type=jax.ShapeDtypeStruct((NW * NPAD,), jnp.float32),
    scratch_types=[
        pltpu.VMEM((BR, LANES), jnp.int32),
        pltpu.VMEM((BR, LANES), jnp.float32),
        pltpu.VMEM((NPAD,), jnp.float32),
        pltpu.SemaphoreType.DMA,
    ],
)
def _deg_kernel(src2, attr2, parts, idx_v, val_v, deg_loc, ld_sem):
    c = lax.axis_index("c")
    s = lax.axis_index("s")
    w = s * 2 + c

    @pl.loop(0, NPAD // 16)
    def _zero(i):
        deg_loc[pl.ds(i * 16, 16)] = jnp.zeros((16,), jnp.float32)

    def scatter_rows(nrows):
        for j in range(nrows):
            for k in range(LANES // 16):
                v = pl.ds(k * 16, 16)
                plsc.addupdate_scatter(deg_loc, [idx_v[j, v]], val_v[j, v])

    def blk(b, carry):
        rb = w * WROWS + b * BR
        h1 = pltpu.async_copy(src2.at[pl.ds(rb, BR)], idx_v, ld_sem)
        h2 = pltpu.async_copy(attr2.at[pl.ds(rb, BR)], val_v, ld_sem)
        h1.wait()
        h2.wait()
        scatter_rows(BR)
        return carry

    nb = jnp.where(w == NW - 1, NB_LAST, NB)
    lax.fori_loop(0, nb, blk, 0)

    @pl.when(w == NW - 1)
    def _tail():
        h1 = pltpu.async_copy(src2.at[pl.ds(TAIL_BASE, TAIL_ROWS)],
                              idx_v.at[pl.ds(0, TAIL_ROWS)], ld_sem)
        h2 = pltpu.async_copy(attr2.at[pl.ds(TAIL_BASE, TAIL_ROWS)],
                              val_v.at[pl.ds(0, TAIL_ROWS)], ld_sem)
        h1.wait()
        h2.wait()
        scatter_rows(TAIL_ROWS)

    pltpu.sync_copy(deg_loc, parts.at[pl.ds(w * NPAD, NPAD)])


@functools.partial(
    pl.kernel,
    mesh=_mesh(),
    compiler_params=_PARAMS,
    out_type=(jax.ShapeDtypeStruct((ROWS, LANES), jnp.float32),
              jax.ShapeDtypeStruct((NPAD,), jnp.float32),
              jax.ShapeDtypeStruct((NPAD,), jnp.float32)),
    scratch_types=[
        pltpu.VMEM((BR, LANES), jnp.int32),
        pltpu.VMEM((BR, LANES), jnp.int32),
        pltpu.VMEM((BR, LANES), jnp.float32),
        pltpu.VMEM((BR, LANES), jnp.float32),
        pltpu.VMEM((NSLICE, SLICE), jnp.float32),
        pltpu.VMEM((SLICE,), jnp.float32),
        pltpu.VMEM((NPAD,), jnp.float32),
        pltpu.SemaphoreType.DMA,
    ],
)
def _norm_kernel(src2, dst2, attr2, parts2, out2, rn0, rn1,
                 si_v, di_v, at_v, out_v, stage, acc_v, rn_full, ld_sem):
    c = lax.axis_index("c")
    s = lax.axis_index("s")
    w = s * 2 + c
    sl = pl.ds(s * SLICE, SLICE)

    pltpu.sync_copy(parts2.at[pl.ds(0, NSLICE), sl], stage)

    @pl.loop(0, SLICE // 16)
    def _pass0(v):
        v16 = pl.ds(v * 16, 16)
        t = stage[0, v16]
        for r in range(1, NSLICE):
            t = t + stage[r, v16]
        acc_v[v16] = t

    pltpu.sync_copy(parts2.at[pl.ds(NSLICE, NSLICE), sl], stage)

    @pl.loop(0, SLICE // 16)
    def _pass1(v):
        v16 = pl.ds(v * 16, 16)
        t = acc_v[v16]
        for r in range(NSLICE):
            t = t + stage[r, v16]
        acc_v[v16] = _rsqrt_nr(t)

    @pl.when(c == 0)
    def _p0():
        pltpu.sync_copy(acc_v, rn0.at[sl])

    @pl.when(c == 1)
    def _p1():
        pltpu.sync_copy(acc_v, rn1.at[sl])

    plsc.subcore_barrier()

    @pl.when(c == 0)
    def _l0():
        pltpu.sync_copy(rn0, rn_full)

    @pl.when(c == 1)
    def _l1():
        pltpu.sync_copy(rn1, rn_full)

    def compute_rows(nrows):
        for j in range(nrows):
            for k in range(LANES // 16):
                v = pl.ds(k * 16, 16)
                r = plsc.load_gather(rn_full, [si_v[j, v]])
                col = plsc.load_gather(rn_full, [di_v[j, v]])
                out_v[j, v] = r * at_v[j, v] * col

    def blk(b, carry):
        rb = w * WROWS + b * BR
        h1 = pltpu.async_copy(src2.at[pl.ds(rb, BR)], si_v, ld_sem)
        h2 = pltpu.async_copy(dst2.at[pl.ds(rb, BR)], di_v, ld_sem)
        h3 = pltpu.async_copy(attr2.at[pl.ds(rb, BR)], at_v, ld_sem)
        h1.wait()
        h2.wait()
        h3.wait()
        compute_rows(BR)
        pltpu.sync_copy(out_v, out2.at[pl.ds(rb, BR)])
        return carry

    nb = jnp.where(w == NW - 1, NB_LAST, NB)
    lax.fori_loop(0, nb, blk, 0)

    @pl.when(w == NW - 1)
    def _tail():
        h1 = pltpu.async_copy(src2.at[pl.ds(TAIL_BASE, TAIL_ROWS)],
                              si_v.at[pl.ds(0, TAIL_ROWS)], ld_sem)
        h2 = pltpu.async_copy(dst2.at[pl.ds(TAIL_BASE, TAIL_ROWS)],
                              di_v.at[pl.ds(0, TAIL_ROWS)], ld_sem)
        h3 = pltpu.async_copy(attr2.at[pl.ds(TAIL_BASE, TAIL_ROWS)],
                              at_v.at[pl.ds(0, TAIL_ROWS)], ld_sem)
        h1.wait()
        h2.wait()
        h3.wait()
        compute_rows(TAIL_ROWS)
        pltpu.sync_copy(out_v.at[pl.ds(0, TAIL_ROWS)],
                        out2.at[pl.ds(TAIL_BASE, TAIL_ROWS)])


def kernel(edge_index, edge_attr, N):
    src2 = edge_index[0].astype(jnp.int32).reshape(ROWS, LANES)
    dst2 = edge_index[1].astype(jnp.int32).reshape(ROWS, LANES)
    attr2 = edge_attr.reshape(ROWS, LANES)
    parts = _deg_kernel(src2, attr2)
    parts2 = parts.reshape(NW, NPAD)
    out2, _, _ = _norm_kernel(src2, dst2, attr2, parts2)
    return out2.reshape(E_STATIC)


# speedup vs baseline: 146.9586x; 1.2960x over previous
import functools

import jax
import jax.numpy as jnp
from jax import lax
from jax.experimental import pallas as pl
from jax.experimental.pallas import tpu as pltpu
from jax.experimental.pallas import tpu_sc as plsc

N_STATIC = 50000
E_STATIC = 1600000
NW = 32
WEDGES = E_STATIC // NW
CH = 2000
NB = WEDGES // CH
NVEC = CH // 16
UNROLL = 5
NSLICE = 16
SLICE = 3200
NPAD = NSLICE * SLICE


def _rsqrt_nr(d):
    i = lax.bitcast_convert_type(d, jnp.int32)
    y = lax.bitcast_convert_type(jnp.int32(0x5F3759DF) - (i >> 1), jnp.float32)
    half = d * jnp.float32(0.5)
    for _ in range(4):
        y = y * (jnp.float32(1.5) - half * y * y)
    return jnp.where(d > jnp.float32(0.0), y, jnp.float32(jnp.inf))


def _mesh():
    return plsc.VectorSubcoreMesh(core_axis_name="c", subcore_axis_name="s")


_PARAMS = pltpu.CompilerParams(needs_layout_passes=False)


@functools.partial(
    pl.kernel,
    mesh=_mesh(),
    compiler_params=_PARAMS,
    out_type=jax.ShapeDtypeStruct((NW * NPAD,), jnp.float32),
    scratch_types=[
        pltpu.VMEM((CH,), jnp.int32),
        pltpu.VMEM((CH,), jnp.int32),
        pltpu.VMEM((CH,), jnp.float32),
        pltpu.VMEM((CH,), jnp.float32),
        pltpu.VMEM((NPAD,), jnp.float32),
        pltpu.SemaphoreType.DMA,
        pltpu.SemaphoreType.DMA,
    ],
)
def _deg_kernel(src1, attr1, parts, idx0, idx1, val0, val1, deg_loc,
                sem0, sem1):
    c = lax.axis_index("c")
    s = lax.axis_index("s")
    w = s * 2 + c
    base = w * WEDGES
    idx_b, val_b, sem_b = (idx0, idx1), (val0, val1), (sem0, sem1)

    def fire(b):
        p = b % 2
        eb = base + b * CH
        return (pltpu.async_copy(src1.at[pl.ds(eb, CH)], idx_b[p], sem_b[p]),
                pltpu.async_copy(attr1.at[pl.ds(eb, CH)], val_b[p], sem_b[p]))

    pend = fire(0)

    @pl.loop(0, NPAD // (16 * 8))
    def _zero(i):
        for u in range(8):
            deg_loc[pl.ds(i * 128 + u * 16, 16)] = jnp.zeros((16,), jnp.float32)

    for b in range(NB):
        nxt = fire(b + 1) if b + 1 < NB else None
        pend[0].wait()
        pend[1].wait()
        p = b % 2
        idx_v, val_v = idx_b[p], val_b[p]

        @pl.loop(0, NVEC // UNROLL)
        def _scatter(i):
            for u in range(UNROLL):
                v = pl.ds(i * (16 * UNROLL) + u * 16, 16)
                plsc.addupdate_scatter(deg_loc, [idx_v[v]], val_v[v])

        pend = nxt

    pltpu.sync_copy(deg_loc, parts.at[pl.ds(w * NPAD, NPAD)])


@functools.partial(
    pl.kernel,
    mesh=_mesh(),
    compiler_params=_PARAMS,
    out_type=(jax.ShapeDtypeStruct((E_STATIC,), jnp.float32),
              jax.ShapeDtypeStruct((NPAD,), jnp.float32),
              jax.ShapeDtypeStruct((NPAD,), jnp.float32)),
    scratch_types=[
        pltpu.VMEM((CH,), jnp.int32),
        pltpu.VMEM((CH,), jnp.int32),
        pltpu.VMEM((CH,), jnp.int32),
        pltpu.VMEM((CH,), jnp.int32),
        pltpu.VMEM((CH,), jnp.float32),
        pltpu.VMEM((CH,), jnp.float32),
        pltpu.VMEM((CH,), jnp.float32),
        pltpu.VMEM((CH,), jnp.float32),
        pltpu.VMEM((NSLICE, SLICE), jnp.float32),
        pltpu.VMEM((SLICE,), jnp.float32),
        pltpu.VMEM((NPAD,), jnp.float32),
        pltpu.SemaphoreType.DMA,
        pltpu.SemaphoreType.DMA,
        pltpu.SemaphoreType.DMA,
        pltpu.SemaphoreType.DMA,
    ],
)
def _norm_kernel(src1, dst1, attr1, parts, out1, rn0, rn1,
                 si0, si1, di0, di1, at0, at1, ou0, ou1,
                 stage, acc_v, rn_full, sem0, sem1, osem0, osem1):
    c = lax.axis_index("c")
    s = lax.axis_index("s")
    w = s * 2 + c
    sl = pl.ds(s * SLICE, SLICE)
    base = w * WEDGES

    def load_pass(p0):
        hs = [pltpu.async_copy(
            parts.at[pl.ds((p0 + r) * NPAD + s * SLICE, SLICE)],
            stage.at[r], sem0) for r in range(NSLICE)]
        for h in hs:
            h.wait()

    load_pass(0)

    @pl.loop(0, SLICE // 16)
    def _pass0(v):
        v16 = pl.ds(v * 16, 16)
        t = stage[0, v16]
        for r in range(1, NSLICE):
            t = t + stage[r, v16]
        acc_v[v16] = t

    load_pass(NSLICE)

    @pl.loop(0, SLICE // 16)
    def _pass1(v):
        v16 = pl.ds(v * 16, 16)
        t = acc_v[v16]
        for r in range(NSLICE):
            t = t + stage[r, v16]
        acc_v[v16] = _rsqrt_nr(t)

    @pl.when(c == 0)
    def _p0():
        pltpu.sync_copy(acc_v, rn0.at[sl])

    @pl.when(c == 1)
    def _p1():
        pltpu.sync_copy(acc_v, rn1.at[sl])

    plsc.subcore_barrier()

    @pl.when(c == 0)
    def _l0():
        pltpu.sync_copy(rn0, rn_full)

    @pl.when(c == 1)
    def _l1():
        pltpu.sync_copy(rn1, rn_full)

    si_b, di_b = (si0, si1), (di0, di1)
    at_b, ou_b, sem_b = (at0, at1), (ou0, ou1), (sem0, sem1)
    osem_b = (osem0, osem1)

    def fire(b):
        p = b % 2
        eb = base + b * CH
        return (pltpu.async_copy(src1.at[pl.ds(eb, CH)], si_b[p], sem_b[p]),
                pltpu.async_copy(dst1.at[pl.ds(eb, CH)], di_b[p], sem_b[p]),
                pltpu.async_copy(attr1.at[pl.ds(eb, CH)], at_b[p], sem_b[p]))

    pend = fire(0)
    opend = [None, None]
    for b in range(NB):
        nxt = fire(b + 1) if b + 1 < NB else None
        for h in pend:
            h.wait()
        p = b % 2
        if opend[p] is not None:
            opend[p].wait()
        si_v, di_v, at_v, ou_v = si_b[p], di_b[p], at_b[p], ou_b[p]

        @pl.loop(0, NVEC // UNROLL)
        def _gather(i):
            for u in range(UNROLL):
                v = pl.ds(i * (16 * UNROLL) + u * 16, 16)
                r = plsc.load_gather(rn_full, [si_v[v]])
                col = plsc.load_gather(rn_full, [di_v[v]])
                ou_v[v] = r * at_v[v] * col

        opend[p] = pltpu.async_copy(ou_v, out1.at[pl.ds(base + b * CH, CH)],
                                    osem_b[p])
        pend = nxt

    for h in opend:
        if h is not None:
            h.wait()


def kernel(edge_index, edge_attr, N):
    src1 = edge_index[0].astype(jnp.int32)
    dst1 = edge_index[1].astype(jnp.int32)
    parts = _deg_kernel(src1, edge_attr)
    out1, _, _ = _norm_kernel(src1, dst1, edge_attr, parts)
    return out1


# speedup vs baseline: 156.9298x; 1.0679x over previous
import functools

import jax
import jax.numpy as jnp
from jax import lax
from jax.experimental import pallas as pl
from jax.experimental.pallas import tpu as pltpu
from jax.experimental.pallas import tpu_sc as plsc

N_STATIC = 50000
E_STATIC = 1600000
NW = 32
WEDGES = E_STATIC // NW
CH = 2000
NB = WEDGES // CH
NVEC = CH // 16
UNROLL = 5
NSLICE = 16
SLICE = 3200
NPAD = NSLICE * SLICE


def _rsqrt_nr(d):
    i = lax.bitcast_convert_type(d, jnp.int32)
    y = lax.bitcast_convert_type(jnp.int32(0x5F3759DF) - (i >> 1), jnp.float32)
    half = d * jnp.float32(0.5)
    for _ in range(4):
        y = y * (jnp.float32(1.5) - half * y * y)
    return jnp.where(d > jnp.float32(0.0), y, jnp.float32(jnp.inf))


@functools.partial(
    pl.kernel,
    mesh=plsc.VectorSubcoreMesh(core_axis_name="c", subcore_axis_name="s"),
    compiler_params=pltpu.CompilerParams(needs_layout_passes=False),
    out_type=(jax.ShapeDtypeStruct((E_STATIC,), jnp.float32),
              jax.ShapeDtypeStruct((NW * NPAD,), jnp.float32),
              jax.ShapeDtypeStruct((NPAD,), jnp.float32),
              jax.ShapeDtypeStruct((NPAD,), jnp.float32)),
    scratch_types=[
        pltpu.VMEM((CH,), jnp.int32),
        pltpu.VMEM((CH,), jnp.int32),
        pltpu.VMEM((CH,), jnp.int32),
        pltpu.VMEM((CH,), jnp.int32),
        pltpu.VMEM((CH,), jnp.float32),
        pltpu.VMEM((CH,), jnp.float32),
        pltpu.VMEM((CH,), jnp.float32),
        pltpu.VMEM((CH,), jnp.float32),
        pltpu.VMEM((NPAD,), jnp.float32),
        pltpu.VMEM((NSLICE, SLICE), jnp.float32),
        pltpu.VMEM((SLICE,), jnp.float32),
        pltpu.SemaphoreType.DMA,
        pltpu.SemaphoreType.DMA,
        pltpu.SemaphoreType.DMA,
        pltpu.SemaphoreType.DMA,
        pltpu.SemaphoreType.DMA,
        pltpu.SemaphoreType.REGULAR,
    ],
)
def _fused_kernel(src1, dst1, attr1, out1, parts, rn0, rn1,
                  si0, si1, di0, di1, at0, at1, ou0, ou1,
                  deg_loc, stage, acc_v,
                  sem0, sem1, osem0, osem1, rsem, gsem):
    c = lax.axis_index("c")
    s = lax.axis_index("s")
    w = s * 2 + c
    sl = pl.ds(s * SLICE, SLICE)
    base = w * WEDGES
    si_b, di_b = (si0, si1), (di0, di1)
    at_b, ou_b = (at0, at1), (ou0, ou1)
    sem_b, osem_b = (sem0, sem1), (osem0, osem1)

    def fire_deg(b):
        p = b % 2
        eb = base + b * CH
        return (pltpu.async_copy(src1.at[pl.ds(eb, CH)], si_b[p], sem_b[p]),
                pltpu.async_copy(attr1.at[pl.ds(eb, CH)], at_b[p], sem_b[p]))

    pend = fire_deg(0)

    @pl.loop(0, NPAD // (16 * 8))
    def _zero(i):
        for u in range(8):
            deg_loc[pl.ds(i * 128 + u * 16, 16)] = jnp.zeros((16,), jnp.float32)

    for b in range(NB):
        nxt = fire_deg(b + 1) if b + 1 < NB else None
        pend[0].wait()
        pend[1].wait()
        p = b % 2
        idx_v, val_v = si_b[p], at_b[p]

        @pl.loop(0, NVEC // UNROLL)
        def _scatter(i):
            for u in range(UNROLL):
                v = pl.ds(i * (16 * UNROLL) + u * 16, 16)
                plsc.addupdate_scatter(deg_loc, [idx_v[v]], val_v[v])

        pend = nxt

    pltpu.sync_copy(deg_loc, parts.at[pl.ds(w * NPAD, NPAD)])

    def fire_norm(b):
        p = b % 2
        eb = base + b * CH
        return (pltpu.async_copy(src1.at[pl.ds(eb, CH)], si_b[p], sem_b[p]),
                pltpu.async_copy(dst1.at[pl.ds(eb, CH)], di_b[p], sem_b[p]),
                pltpu.async_copy(attr1.at[pl.ds(eb, CH)], at_b[p], sem_b[p]))

    pend = fire_norm(0)

    plsc.subcore_barrier()
    pltpu.core_barrier(gsem, core_axis_name="c")

    def load_pass(p0):
        hs = [pltpu.async_copy(
            parts.at[pl.ds((p0 + r) * NPAD + s * SLICE, SLICE)],
            stage.at[r], rsem) for r in range(NSLICE)]
        for h in hs:
            h.wait()

    load_pass(0)

    @pl.loop(0, SLICE // 16)
    def _pass0(v):
        v16 = pl.ds(v * 16, 16)
        t = stage[0, v16]
        for r in range(1, NSLICE):
            t = t + stage[r, v16]
        acc_v[v16] = t

    load_pass(NSLICE)

    @pl.loop(0, SLICE // 16)
    def _pass1(v):
        v16 = pl.ds(v * 16, 16)
        t = acc_v[v16]
        for r in range(NSLICE):
            t = t + stage[r, v16]
        acc_v[v16] = _rsqrt_nr(t)

    @pl.when(c == 0)
    def _p0():
        pltpu.sync_copy(acc_v, rn0.at[sl])

    @pl.when(c == 1)
    def _p1():
        pltpu.sync_copy(acc_v, rn1.at[sl])

    plsc.subcore_barrier()

    rn_full = deg_loc

    @pl.when(c == 0)
    def _l0():
        pltpu.sync_copy(rn0, rn_full)

    @pl.when(c == 1)
    def _l1():
        pltpu.sync_copy(rn1, rn_full)

    opend = [None, None]
    for b in range(NB):
        nxt = fire_norm(b + 1) if b + 1 < NB else None
        for h in pend:
            h.wait()
        p = b % 2
        if opend[p] is not None:
            opend[p].wait()
        si_v, di_v, at_v, ou_v = si_b[p], di_b[p], at_b[p], ou_b[p]

        @pl.loop(0, NVEC // UNROLL)
        def _gather(i):
            for u in range(UNROLL):
                v = pl.ds(i * (16 * UNROLL) + u * 16, 16)
                r = plsc.load_gather(rn_full, [si_v[v]])
                col = plsc.load_gather(rn_full, [di_v[v]])
                ou_v[v] = r * at_v[v] * col

        opend[p] = pltpu.async_copy(ou_v, out1.at[pl.ds(base + b * CH, CH)],
                                    osem_b[p])
        pend = nxt

    for h in opend:
        if h is not None:
            h.wait()


def kernel(edge_index, edge_attr, N):
    src1 = edge_index[0].astype(jnp.int32)
    dst1 = edge_index[1].astype(jnp.int32)
    out1, _, _, _ = _fused_kernel(src1, dst1, edge_attr)
    return out1


# speedup vs baseline: 187.1025x; 1.1923x over previous
import functools

import jax
import jax.numpy as jnp
from jax import lax
from jax.experimental import pallas as pl
from jax.experimental.pallas import tpu as pltpu
from jax.experimental.pallas import tpu_sc as plsc

N_STATIC = 50000
E_STATIC = 1600000
NW = 32
WEDGES = E_STATIC // NW
CH = 2000
NB = WEDGES // CH
NVEC = CH // 16
UNROLL = 5
NSLICE = 16
SLICE = 3200
NPAD = NSLICE * SLICE


def _rsqrt_nr(d):
    i = lax.bitcast_convert_type(d, jnp.int32)
    y = lax.bitcast_convert_type(jnp.int32(0x5F3759DF) - (i >> 1), jnp.float32)
    half = d * jnp.float32(0.5)
    for _ in range(4):
        y = y * (jnp.float32(1.5) - half * y * y)
    return jnp.where(d > jnp.float32(0.0), y, jnp.float32(jnp.inf))


def _mesh():
    return plsc.VectorSubcoreMesh(core_axis_name="c", subcore_axis_name="s")


_PARAMS = pltpu.CompilerParams(needs_layout_passes=False)


@functools.partial(
    pl.kernel,
    mesh=_mesh(),
    compiler_params=_PARAMS,
    out_type=jax.ShapeDtypeStruct((NW * NPAD,), jnp.float32),
    scratch_types=[
        pltpu.VMEM((CH,), jnp.int32),
        pltpu.VMEM((CH,), jnp.int32),
        pltpu.VMEM((CH,), jnp.float32),
        pltpu.VMEM((CH,), jnp.float32),
        pltpu.VMEM((NPAD,), jnp.float32),
        pltpu.SemaphoreType.DMA,
        pltpu.SemaphoreType.DMA,
    ],
)
def _deg_kernel(ei1, attr1, parts, idx0, idx1, val0, val1, deg_loc,
                sem0, sem1):
    c = lax.axis_index("c")
    s = lax.axis_index("s")
    w = s * 2 + c
    base = w * WEDGES
    idx_b, val_b, sem_b = (idx0, idx1), (val0, val1), (sem0, sem1)

    def fire(b):
        p = b % 2
        eb = base + b * CH
        return (pltpu.async_copy(ei1.at[pl.ds(eb, CH)], idx_b[p], sem_b[p]),
                pltpu.async_copy(attr1.at[pl.ds(eb, CH)], val_b[p], sem_b[p]))

    pend = fire(0)

    @pl.loop(0, NPAD // (16 * 8))
    def _zero(i):
        for u in range(8):
            deg_loc[pl.ds(i * 128 + u * 16, 16)] = jnp.zeros((16,), jnp.float32)

    for b in range(NB):
        nxt = fire(b + 1) if b + 1 < NB else None
        pend[0].wait()
        pend[1].wait()
        p = b % 2
        idx_v, val_v = idx_b[p], val_b[p]

        @pl.loop(0, NVEC // UNROLL)
        def _scatter(i):
            for u in range(UNROLL):
                v = pl.ds(i * (16 * UNROLL) + u * 16, 16)
                plsc.addupdate_scatter(deg_loc, [idx_v[v]], val_v[v])

        pend = nxt

    pltpu.sync_copy(deg_loc, parts.at[pl.ds(w * NPAD, NPAD)])


@functools.partial(
    pl.kernel,
    mesh=_mesh(),
    compiler_params=_PARAMS,
    out_type=(jax.ShapeDtypeStruct((E_STATIC,), jnp.float32),
              jax.ShapeDtypeStruct((NPAD,), jnp.float32),
              jax.ShapeDtypeStruct((NPAD,), jnp.float32)),
    scratch_types=[
        pltpu.VMEM((CH,), jnp.int32),
        pltpu.VMEM((CH,), jnp.int32),
        pltpu.VMEM((CH,), jnp.int32),
        pltpu.VMEM((CH,), jnp.int32),
        pltpu.VMEM((CH,), jnp.float32),
        pltpu.VMEM((CH,), jnp.float32),
        pltpu.VMEM((CH,), jnp.float32),
        pltpu.VMEM((CH,), jnp.float32),
        pltpu.VMEM((NSLICE, SLICE), jnp.float32),
        pltpu.VMEM((SLICE,), jnp.float32),
        pltpu.VMEM((NPAD,), jnp.float32),
        pltpu.SemaphoreType.DMA,
        pltpu.SemaphoreType.DMA,
        pltpu.SemaphoreType.DMA,
        pltpu.SemaphoreType.DMA,
    ],
)
def _norm_kernel(ei1, attr1, parts, out1, rn0, rn1,
                 si0, si1, di0, di1, at0, at1, ou0, ou1,
                 stage, acc_v, rn_full, sem0, sem1, osem0, osem1):
    c = lax.axis_index("c")
    s = lax.axis_index("s")
    w = s * 2 + c
    sl = pl.ds(s * SLICE, SLICE)
    base = w * WEDGES

    def load_pass(p0):
        hs = [pltpu.async_copy(
            parts.at[pl.ds((p0 + r) * NPAD + s * SLICE, SLICE)],
            stage.at[r], sem0) for r in range(NSLICE)]
        for h in hs:
            h.wait()

    load_pass(0)

    @pl.loop(0, SLICE // 16)
    def _pass0(v):
        v16 = pl.ds(v * 16, 16)
        t = stage[0, v16]
        for r in range(1, NSLICE):
            t = t + stage[r, v16]
        acc_v[v16] = t

    load_pass(NSLICE)

    @pl.loop(0, SLICE // 16)
    def _pass1(v):
        v16 = pl.ds(v * 16, 16)
        t = acc_v[v16]
        for r in range(NSLICE):
            t = t + stage[r, v16]
        acc_v[v16] = _rsqrt_nr(t)

    @pl.when(c == 0)
    def _p0():
        pltpu.sync_copy(acc_v, rn0.at[sl])

    @pl.when(c == 1)
    def _p1():
        pltpu.sync_copy(acc_v, rn1.at[sl])

    plsc.subcore_barrier()

    @pl.when(c == 0)
    def _l0():
        pltpu.sync_copy(rn0, rn_full)

    @pl.when(c == 1)
    def _l1():
        pltpu.sync_copy(rn1, rn_full)

    si_b, di_b = (si0, si1), (di0, di1)
    at_b, ou_b, sem_b = (at0, at1), (ou0, ou1), (sem0, sem1)
    osem_b = (osem0, osem1)

    def fire(b):
        p = b % 2
        eb = base + b * CH
        return (pltpu.async_copy(ei1.at[pl.ds(eb, CH)], si_b[p], sem_b[p]),
                pltpu.async_copy(ei1.at[pl.ds(E_STATIC + eb, CH)], di_b[p],
                                 sem_b[p]),
                pltpu.async_copy(attr1.at[pl.ds(eb, CH)], at_b[p], sem_b[p]))

    pend = fire(0)
    opend = [None, None]
    for b in range(NB):
        nxt = fire(b + 1) if b + 1 < NB else None
        for h in pend:
            h.wait()
        p = b % 2
        if opend[p] is not None:
            opend[p].wait()
        si_v, di_v, at_v, ou_v = si_b[p], di_b[p], at_b[p], ou_b[p]

        @pl.loop(0, NVEC // UNROLL)
        def _gather(i):
            for u in range(UNROLL):
                v = pl.ds(i * (16 * UNROLL) + u * 16, 16)
                r = plsc.load_gather(rn_full, [si_v[v]])
                col = plsc.load_gather(rn_full, [di_v[v]])
                ou_v[v] = r * at_v[v] * col

        opend[p] = pltpu.async_copy(ou_v, out1.at[pl.ds(base + b * CH, CH)],
                                    osem_b[p])
        pend = nxt

    for h in opend:
        if h is not None:
            h.wait()


def kernel(edge_index, edge_attr, N):
    ei1 = edge_index.astype(jnp.int32).reshape(2 * E_STATIC)
    parts = _deg_kernel(ei1, edge_attr)
    out1, _, _ = _norm_kernel(ei1, edge_attr, parts)
    return out1


# speedup vs baseline: 217.8910x; 1.1646x over previous
import functools

import jax
import jax.numpy as jnp
from jax import lax
from jax.experimental import pallas as pl
from jax.experimental.pallas import tpu as pltpu
from jax.experimental.pallas import tpu_sc as plsc

N_STATIC = 50000
E_STATIC = 1600000
NW = 32
CH = 2048
NCH = E_STATIC // CH
PCH = E_STATIC - NCH * CH
NB = NCH // NW
NEXTRA = NCH - NB * NW
UNROLL = 8
NSLICE = 16
SLICE = 3200
NPAD = NSLICE * SLICE


def _rsqrt_nr(d):
    i = lax.bitcast_convert_type(d, jnp.int32)
    y = lax.bitcast_convert_type(jnp.int32(0x5F3759DF) - (i >> 1), jnp.float32)
    half = d * jnp.float32(0.5)
    for _ in range(4):
        y = y * (jnp.float32(1.5) - half * y * y)
    return jnp.where(d > jnp.float32(0.0), y, jnp.float32(jnp.inf))


def _mesh():
    return plsc.VectorSubcoreMesh(core_axis_name="c", subcore_axis_name="s")


_PARAMS = pltpu.CompilerParams(needs_layout_passes=False)


@functools.partial(
    pl.kernel,
    mesh=_mesh(),
    compiler_params=_PARAMS,
    out_type=jax.ShapeDtypeStruct((NW * NPAD,), jnp.float32),
    scratch_types=[
        pltpu.VMEM((2, CH), jnp.int32),
        pltpu.VMEM((2, CH), jnp.int32),
        pltpu.VMEM((CH,), jnp.float32),
        pltpu.VMEM((CH,), jnp.float32),
        pltpu.VMEM((NPAD,), jnp.float32),
        pltpu.SemaphoreType.DMA,
        pltpu.SemaphoreType.DMA,
    ],
)
def _deg_kernel(ei2, attr1, parts, ib0, ib1, ab0, ab1, deg_loc, sem0, sem1):
    c = lax.axis_index("c")
    s = lax.axis_index("s")
    w = s * 2 + c
    ib_b, ab_b, sem_b = (ib0, ib1), (ab0, ab1), (sem0, sem1)

    def fire(b):
        p = b % 2
        eb = jnp.minimum(w + b * NW, NCH - 1) * CH
        return (pltpu.async_copy(ei2.at[:, pl.ds(eb, CH)], ib_b[p], sem_b[p]),
                pltpu.async_copy(attr1.at[pl.ds(eb, CH)], ab_b[p], sem_b[p]))

    pend = fire(0)

    @pl.loop(0, NPAD // (16 * 8))
    def _zero(i):
        for u in range(8):
            deg_loc[pl.ds(i * 128 + u * 16, 16)] = jnp.zeros((16,), jnp.float32)

    def scatter(ib_v, ab_v, nvec):
        @pl.loop(0, nvec // UNROLL)
        def _scatter(i):
            for u in range(UNROLL):
                v = pl.ds(i * (16 * UNROLL) + u * 16, 16)
                plsc.addupdate_scatter(deg_loc, [ib_v[0, v]], ab_v[v])

    for b in range(NB):
        nxt = fire(b + 1) if b + 1 < NB + 1 else None
        pend[0].wait()
        pend[1].wait()
        p = b % 2
        scatter(ib_b[p], ab_b[p], CH // 16)
        pend = nxt

    pend[0].wait()
    pend[1].wait()

    @pl.when(w < NEXTRA)
    def _extra():
        scatter(ib_b[NB % 2], ab_b[NB % 2], CH // 16)

    @pl.when(w == NW - 1)
    def _partial():
        eb = NCH * CH
        pltpu.sync_copy(ei2.at[:, pl.ds(eb, PCH)],
                        ib0.at[:, pl.ds(0, PCH)])
        pltpu.sync_copy(attr1.at[pl.ds(eb, PCH)], ab0.at[pl.ds(0, PCH)])
        scatter(ib0, ab0, PCH // 16)

    pltpu.sync_copy(deg_loc, parts.at[pl.ds(w * NPAD, NPAD)])


@functools.partial(
    pl.kernel,
    mesh=_mesh(),
    compiler_params=_PARAMS,
    out_type=jax.ShapeDtypeStruct((E_STATIC,), jnp.float32),
    scratch_types=[
        pltpu.VMEM((2, CH), jnp.int32),
        pltpu.VMEM((2, CH), jnp.int32),
        pltpu.VMEM((CH,), jnp.float32),
        pltpu.VMEM((CH,), jnp.float32),
        pltpu.VMEM((CH,), jnp.float32),
        pltpu.VMEM((CH,), jnp.float32),
        pltpu.VMEM((NSLICE, SLICE), jnp.float32),
        pltpu.VMEM((SLICE,), jnp.float32),
        pltpu.VMEM((NPAD,), jnp.float32),
        pltpu.VMEM_SHARED((NPAD,), jnp.float32),
        pltpu.SemaphoreType.DMA,
        pltpu.SemaphoreType.DMA,
        pltpu.SemaphoreType.DMA,
        pltpu.SemaphoreType.DMA,
    ],
)
def _norm_kernel(ei2, attr1, parts, out1,
                 ib0, ib1, ab0, ab1, ob0, ob1,
                 stage, acc_v, rn_full, rn_sh, sem0, sem1, osem0, osem1):
    c = lax.axis_index("c")
    s = lax.axis_index("s")
    w = s * 2 + c
    sl = pl.ds(s * SLICE, SLICE)

    def load_pass(p0):
        hs = [pltpu.async_copy(
            parts.at[pl.ds((p0 + r) * NPAD + s * SLICE, SLICE)],
            stage.at[r], sem0) for r in range(NSLICE)]
        for h in hs:
            h.wait()

    load_pass(0)

    @pl.loop(0, SLICE // 16)
    def _pass0(v):
        v16 = pl.ds(v * 16, 16)
        t = stage[0, v16]
        for r in range(1, NSLICE):
            t = t + stage[r, v16]
        acc_v[v16] = t

    load_pass(NSLICE)

    @pl.loop(0, SLICE // 16)
    def _pass1(v):
        v16 = pl.ds(v * 16, 16)
        t = acc_v[v16]
        for r in range(NSLICE):
            t = t + stage[r, v16]
        acc_v[v16] = _rsqrt_nr(t)

    pltpu.sync_copy(acc_v, rn_sh.at[sl])
    plsc.subcore_barrier()
    pltpu.sync_copy(rn_sh, rn_full)

    ib_b, ab_b = (ib0, ib1), (ab0, ab1)
    ob_b, sem_b, osem_b = (ob0, ob1), (sem0, sem1), (osem0, osem1)

    def fire(b):
        p = b % 2
        eb = jnp.minimum(w + b * NW, NCH - 1) * CH
        return (pltpu.async_copy(ei2.at[:, pl.ds(eb, CH)], ib_b[p], sem_b[p]),
                pltpu.async_copy(attr1.at[pl.ds(eb, CH)], ab_b[p], sem_b[p]))

    def gather(ib_v, ab_v, ob_v, nvec):
        @pl.loop(0, nvec // UNROLL)
        def _gather(i):
            for u in range(UNROLL):
                v = pl.ds(i * (16 * UNROLL) + u * 16, 16)
                r = plsc.load_gather(rn_full, [ib_v[0, v]])
                col = plsc.load_gather(rn_full, [ib_v[1, v]])
                ob_v[v] = r * ab_v[v] * col

    pend = fire(0)
    opend = [None, None]
    for b in range(NB):
        nxt = fire(b + 1) if b + 1 < NB + 1 else None
        pend[0].wait()
        pend[1].wait()
        p = b % 2
        if opend[p] is not None:
            opend[p].wait()
        gather(ib_b[p], ab_b[p], ob_b[p], CH // 16)
        opend[p] = pltpu.async_copy(
            ob_b[p], out1.at[pl.ds((w + b * NW) * CH, CH)], osem_b[p])
        pend = nxt

    pend[0].wait()
    pend[1].wait()
    for h in opend:
        if h is not None:
            h.wait()

    @pl.when(w < NEXTRA)
    def _extra():
        p = NB % 2
        gather(ib_b[p], ab_b[p], ob_b[p], CH // 16)
        pltpu.sync_copy(ob_b[p], out1.at[pl.ds((w + NB * NW) * CH, CH)])

    @pl.when(w == NW - 1)
    def _partial():
        eb = NCH * CH
        pltpu.sync_copy(ei2.at[:, pl.ds(eb, PCH)], ib0.at[:, pl.ds(0, PCH)])
        pltpu.sync_copy(attr1.at[pl.ds(eb, PCH)], ab0.at[pl.ds(0, PCH)])
        gather(ib0, ab0, ob0, PCH // 16)
        pltpu.sync_copy(ob0.at[pl.ds(0, PCH)], out1.at[pl.ds(eb, PCH)])


def kernel(edge_index, edge_attr, N):
    ei2 = edge_index.astype(jnp.int32)
    parts = _deg_kernel(ei2, edge_attr)
    return _norm_kernel(ei2, edge_attr, parts)
